# Initial kernel scaffold; baseline (speedup 1.0000x reference)
import jax
import jax.numpy as jnp
from jax.experimental import pallas as pl


def kernel(x, edge_index, edge_attr, W1, b1, W2, b2):
    raise NotImplementedError("write your pallas kernel here")


# speedup vs baseline: 41.6917x; 41.6917x over previous
import functools

import jax
import jax.numpy as jnp
from jax import lax
from jax.experimental import pallas as pl
from jax.experimental.pallas import tpu as pltpu
from jax.experimental.pallas import tpu_sc as plsc

N = 10000
E = 320000
D = 128
H = 16
C = 40

NC = 2
NS = 16
NW = NC * NS
EPT = E // NW
CHUNK = 1000
NCHUNK = EPT // CHUNK
RPT = N // NS
UNROLL = 8

_mesh = plsc.VectorSubcoreMesh(core_axis_name="c", subcore_axis_name="s")
_sc_params = pltpu.CompilerParams(use_tc_tiling_on_sc=False)


def _scale_rows(w_v, rows_v, count):
    def body(g, carry):
        wv = w_v[pl.ds(g * 16, 16)]
        for j in range(16):
            e = g * 16 + j
            wsplat = lax.gather(
                wv, jnp.full((16, 1), j, jnp.int32),
                lax.GatherDimensionNumbers(offset_dims=(),
                                           collapsed_slice_dims=(0,),
                                           start_index_map=(0,)),
                (1,), mode=lax.GatherScatterMode.PROMISE_IN_BOUNDS)
            rows_v[e, :] = rows_v[e, :] * wsplat
        return carry
    lax.fori_loop(0, count // 16, body, 0)


@functools.partial(
    pl.kernel,
    out_type=jax.ShapeDtypeStruct((NC * N, H), jnp.float32),
    mesh=_mesh,
    scratch_types=[
        pltpu.VMEM((CHUNK,), jnp.int32),
        pltpu.VMEM((CHUNK,), jnp.int32),
        pltpu.VMEM((CHUNK,), jnp.float32),
        pltpu.VMEM((CHUNK, H), jnp.float32),
        pltpu.VMEM((1000, H), jnp.float32),
        pltpu.VMEM_SHARED((N, H), jnp.float32),
        pltpu.SemaphoreType.DMA,
    ],
    compiler_params=_sc_params,
)
def _sc_aggregate(p_hbm, src_hbm, dst_hbm, w_hbm, out_hbm,
                  src_v, dst_v, w_v, rows_v, zrow_v, acc_sp, sem):
    c = lax.axis_index("c")
    s = lax.axis_index("s")
    wid = c * NS + s

    @pl.when(s < 10)
    def _():
        def zbody(i, carry):
            zrow_v[i, :] = jnp.zeros((H,), jnp.float32)
            return carry
        lax.fori_loop(0, 1000, zbody, 0)
        pltpu.sync_copy(zrow_v, acc_sp.at[pl.ds(s * 1000, 1000)])
    plsc.subcore_barrier()

    for k in range(NCHUNK):
        base = wid * EPT + k * CHUNK
        pltpu.sync_copy(src_hbm.at[pl.ds(base, CHUNK)], src_v)
        pltpu.sync_copy(dst_hbm.at[pl.ds(base, CHUNK)], dst_v)
        pltpu.sync_copy(w_hbm.at[pl.ds(base, CHUNK)], w_v)
        pltpu.async_copy(p_hbm.at[src_v], rows_v, sem).wait()
        _scale_rows(w_v, rows_v, CHUNK)
        pltpu.sync_copy(rows_v, acc_sp.at[dst_v], add=True)

    plsc.subcore_barrier()

    @pl.when(s < 10)
    def _():
        pltpu.sync_copy(acc_sp.at[pl.ds(s * 1000, 1000)], zrow_v)
        pltpu.sync_copy(zrow_v, out_hbm.at[pl.ds(c * N + s * 1000, 1000)])


@functools.partial(
    pl.kernel,
    out_type=jax.ShapeDtypeStruct((NC * N,), jnp.float32),
    mesh=_mesh,
    scratch_types=[
        pltpu.VMEM((CHUNK,), jnp.int32),
        pltpu.VMEM((CHUNK,), jnp.float32),
        pltpu.VMEM((1024,), jnp.float32),
        pltpu.VMEM_SHARED((N,), jnp.float32),
    ],
    compiler_params=_sc_params,
)
def _sc_degree(dst_hbm, w_hbm, out_hbm, dst_v, w_v, zbuf_v, acc_sp):
    c = lax.axis_index("c")
    s = lax.axis_index("s")
    wid = c * NS + s

    def zbody(i, carry):
        zbuf_v[pl.ds(i * 16, 16)] = jnp.zeros((16,), jnp.float32)
        return carry
    lax.fori_loop(0, 64, zbody, 0)

    @pl.when(s < 10)
    def _():
        pltpu.sync_copy(zbuf_v.at[pl.ds(0, 1000)],
                        acc_sp.at[pl.ds(s * 1000, 1000)])
    plsc.subcore_barrier()

    for k in range(NCHUNK):
        base = wid * EPT + k * CHUNK
        pltpu.sync_copy(dst_hbm.at[pl.ds(base, CHUNK)], dst_v)
        pltpu.sync_copy(w_hbm.at[pl.ds(base, CHUNK)], w_v)
        pltpu.sync_copy(w_v, acc_sp.at[dst_v], add=True)

    plsc.subcore_barrier()

    @pl.when(s < 10)
    def _():
        pltpu.sync_copy(acc_sp.at[pl.ds(s * 1000, 1000)],
                        zbuf_v.at[pl.ds(0, 1000)])
        pltpu.sync_copy(zbuf_v.at[pl.ds(0, 1000)],
                        out_hbm.at[pl.ds(c * N + s * 1000, 1000)])


def _tc_prep_body(x_ref, w1_ref, degp_ref, h1_ref, p1_ref, dinv_ref):
    h = jnp.dot(x_ref[:], w1_ref[:], preferred_element_type=jnp.float32)
    deg = degp_ref[0, :] + degp_ref[1, :] + 1.0
    dinv = lax.rsqrt(deg)
    dinv2d = jnp.broadcast_to(dinv[:, None], (N, H))
    h1_ref[:] = h
    p1_ref[:] = dinv2d * h
    dinv_ref[:] = dinv2d


def _tc_mid_body(aggp_ref, h1_ref, dinv_ref, b1_ref, r_ref, p2_ref):
    dinv = dinv_ref[:]
    agg = aggp_ref[0] + aggp_ref[1]
    s1 = dinv * agg + dinv * dinv * h1_ref[:] + b1_ref[:]
    r = jnp.maximum(s1, 0.0)
    r_ref[:] = r
    p2_ref[:] = dinv * r


def _tc_out_body(aggp_ref, r_ref, dinv_ref, w2_ref, b2_ref, o_ref):
    dinv = dinv_ref[:]
    agg = aggp_ref[0] + aggp_ref[1]
    s2 = dinv * agg + dinv * dinv * r_ref[:]
    z = jnp.dot(s2, w2_ref[:], preferred_element_type=jnp.float32) + b2_ref[:]
    m = jnp.max(z, axis=1, keepdims=True)
    ez = jnp.exp(z - m)
    lse = jnp.log(jnp.sum(ez, axis=1, keepdims=True)) + m
    o_ref[:] = z - lse


_tc_prep = pl.pallas_call(
    _tc_prep_body,
    out_shape=(
        jax.ShapeDtypeStruct((N, H), jnp.float32),
        jax.ShapeDtypeStruct((N, H), jnp.float32),
        jax.ShapeDtypeStruct((N, H), jnp.float32),
    ),
)

_tc_mid = pl.pallas_call(
    _tc_mid_body,
    out_shape=(
        jax.ShapeDtypeStruct((N, H), jnp.float32),
        jax.ShapeDtypeStruct((N, H), jnp.float32),
    ),
)

_tc_out = pl.pallas_call(
    _tc_out_body,
    out_shape=jax.ShapeDtypeStruct((N, C), jnp.float32),
)


@jax.jit
def kernel(x, edge_index, edge_attr, W1, b1, W2, b2):
    src = edge_index[0]
    dst = edge_index[1]
    degp = _sc_degree(dst, edge_attr).reshape(NC, N)
    h1, p1, dinv = _tc_prep(x, W1, degp)
    agg1 = _sc_aggregate(p1, src, dst, edge_attr).reshape(NC, N, H)
    r, p2 = _tc_mid(agg1, h1, dinv, b1.reshape(1, H))
    agg2 = _sc_aggregate(p2, src, dst, edge_attr).reshape(NC, N, H)
    return _tc_out(agg2, r, dinv, W2, b2.reshape(1, C))


# speedup vs baseline: 56.4207x; 1.3533x over previous
import functools

import jax
import jax.numpy as jnp
from jax import lax
from jax.experimental import pallas as pl
from jax.experimental.pallas import tpu as pltpu
from jax.experimental.pallas import tpu_sc as plsc

N = 10000
E = 320000
D = 128
H = 16
C = 40

NC = 2
NS = 16
NW = NC * NS
EPT = E // NW
CHUNK = 1000
NCHUNK = EPT // CHUNK
RPT = N // NS
UNROLL = 8

_mesh = plsc.VectorSubcoreMesh(core_axis_name="c", subcore_axis_name="s")
_sc_params = pltpu.CompilerParams(use_tc_tiling_on_sc=False)


def _scale_rows(w_v, rows_v, count):
    def body(g, carry):
        wv = w_v[pl.ds(g * 16, 16)]
        for j in range(16):
            e = g * 16 + j
            wsplat = lax.gather(
                wv, jnp.full((16, 1), j, jnp.int32),
                lax.GatherDimensionNumbers(offset_dims=(),
                                           collapsed_slice_dims=(0,),
                                           start_index_map=(0,)),
                (1,), mode=lax.GatherScatterMode.PROMISE_IN_BOUNDS)
            rows_v[e, :] = rows_v[e, :] * wsplat
        return carry
    lax.fori_loop(0, count // 16, body, 0)


@functools.partial(
    pl.kernel,
    out_type=jax.ShapeDtypeStruct((NC * N, H), jnp.float32),
    mesh=_mesh,
    scratch_types=[
        pltpu.VMEM((CHUNK,), jnp.int32),
        pltpu.VMEM((CHUNK,), jnp.int32),
        pltpu.VMEM((CHUNK,), jnp.int32),
        pltpu.VMEM((CHUNK,), jnp.int32),
        pltpu.VMEM((CHUNK,), jnp.float32),
        pltpu.VMEM((CHUNK,), jnp.float32),
        pltpu.VMEM((CHUNK, H), jnp.float32),
        pltpu.VMEM((CHUNK, H), jnp.float32),
        pltpu.VMEM((1000, H), jnp.float32),
        pltpu.VMEM_SHARED((N, H), jnp.float32),
        pltpu.SemaphoreType.DMA,
        pltpu.SemaphoreType.DMA,
        pltpu.SemaphoreType.DMA,
        pltpu.SemaphoreType.DMA,
        pltpu.SemaphoreType.DMA,
        pltpu.SemaphoreType.DMA,
    ],
    compiler_params=_sc_params,
)
def _sc_aggregate(p_hbm, src_hbm, dst_hbm, w_hbm, out_hbm,
                  src_v0, src_v1, dst_v0, dst_v1, w_v0, w_v1,
                  rows_v0, rows_v1, zrow_v, acc_sp,
                  isem0, isem1, gsem0, gsem1, ssem0, ssem1):
    c = lax.axis_index("c")
    s = lax.axis_index("s")
    wid = c * NS + s
    src_v = (src_v0, src_v1)
    dst_v = (dst_v0, dst_v1)
    w_v = (w_v0, w_v1)
    rows_v = (rows_v0, rows_v1)
    isem = (isem0, isem1)
    gsem = (gsem0, gsem1)
    ssem = (ssem0, ssem1)

    @pl.when(s < 10)
    def _():
        def zbody(i, carry):
            zrow_v[i, :] = jnp.zeros((H,), jnp.float32)
            return carry
        lax.fori_loop(0, 1000, zbody, 0)
        pltpu.sync_copy(zrow_v, acc_sp.at[pl.ds(s * 1000, 1000)])
    plsc.subcore_barrier()

    def start_idx(k):
        b = k % 2
        base = wid * EPT + k * CHUNK
        pltpu.async_copy(src_hbm.at[pl.ds(base, CHUNK)], src_v[b], isem[b])
        pltpu.async_copy(dst_hbm.at[pl.ds(base, CHUNK)], dst_v[b], isem[b])
        pltpu.async_copy(w_hbm.at[pl.ds(base, CHUNK)], w_v[b], isem[b])

    def wait_idx(k):
        b = k % 2
        pltpu.make_async_copy(src_hbm.at[pl.ds(0, CHUNK)], src_v[b],
                              isem[b]).wait()
        pltpu.make_async_copy(dst_hbm.at[pl.ds(0, CHUNK)], dst_v[b],
                              isem[b]).wait()
        pltpu.make_async_copy(w_hbm.at[pl.ds(0, CHUNK)], w_v[b],
                              isem[b]).wait()

    def start_gather(k):
        b = k % 2
        pltpu.async_copy(p_hbm.at[src_v[b]], rows_v[b], gsem[b])

    def wait_gather(k):
        b = k % 2
        pltpu.make_async_copy(p_hbm.at[src_v[b]], rows_v[b], gsem[b]).wait()

    def start_scatter(k):
        b = k % 2
        pltpu.async_copy(rows_v[b], acc_sp.at[dst_v[b]], ssem[b], add=True)

    def wait_scatter(k):
        b = k % 2
        pltpu.make_async_copy(rows_v[b], acc_sp.at[dst_v[b]], ssem[b]).wait()

    start_idx(0)
    wait_idx(0)
    start_gather(0)
    for k in range(NCHUNK):
        b = k % 2
        if k + 1 < NCHUNK:
            if k >= 1:
                wait_scatter(k - 1)
            start_idx(k + 1)
        wait_gather(k)
        if k + 1 < NCHUNK:
            wait_idx(k + 1)
            start_gather(k + 1)
        _scale_rows(w_v[b], rows_v[b], CHUNK)
        start_scatter(k)
    wait_scatter(NCHUNK - 2)
    wait_scatter(NCHUNK - 1)

    plsc.subcore_barrier()

    @pl.when(s < 10)
    def _():
        pltpu.sync_copy(acc_sp.at[pl.ds(s * 1000, 1000)], zrow_v)
        pltpu.sync_copy(zrow_v, out_hbm.at[pl.ds(c * N + s * 1000, 1000)])


@functools.partial(
    pl.kernel,
    out_type=jax.ShapeDtypeStruct((NC * N,), jnp.float32),
    mesh=_mesh,
    scratch_types=[
        pltpu.VMEM((CHUNK,), jnp.int32),
        pltpu.VMEM((CHUNK,), jnp.int32),
        pltpu.VMEM((CHUNK,), jnp.float32),
        pltpu.VMEM((CHUNK,), jnp.float32),
        pltpu.VMEM((1024,), jnp.float32),
        pltpu.VMEM_SHARED((N,), jnp.float32),
        pltpu.SemaphoreType.DMA,
        pltpu.SemaphoreType.DMA,
        pltpu.SemaphoreType.DMA,
        pltpu.SemaphoreType.DMA,
    ],
    compiler_params=_sc_params,
)
def _sc_degree(dst_hbm, w_hbm, out_hbm, dst_v0, dst_v1, w_v0, w_v1,
               zbuf_v, acc_sp, isem0, isem1, ssem0, ssem1):
    c = lax.axis_index("c")
    s = lax.axis_index("s")
    wid = c * NS + s
    dst_v = (dst_v0, dst_v1)
    w_v = (w_v0, w_v1)
    isem = (isem0, isem1)
    ssem = (ssem0, ssem1)

    def zbody(i, carry):
        zbuf_v[pl.ds(i * 16, 16)] = jnp.zeros((16,), jnp.float32)
        return carry
    lax.fori_loop(0, 64, zbody, 0)

    @pl.when(s < 10)
    def _():
        pltpu.sync_copy(zbuf_v.at[pl.ds(0, 1000)],
                        acc_sp.at[pl.ds(s * 1000, 1000)])
    plsc.subcore_barrier()

    def start_idx(k):
        b = k % 2
        base = wid * EPT + k * CHUNK
        pltpu.async_copy(dst_hbm.at[pl.ds(base, CHUNK)], dst_v[b], isem[b])
        pltpu.async_copy(w_hbm.at[pl.ds(base, CHUNK)], w_v[b], isem[b])

    def wait_idx(k):
        b = k % 2
        pltpu.make_async_copy(dst_hbm.at[pl.ds(0, CHUNK)], dst_v[b],
                              isem[b]).wait()
        pltpu.make_async_copy(w_hbm.at[pl.ds(0, CHUNK)], w_v[b],
                              isem[b]).wait()

    def start_scatter(k):
        b = k % 2
        pltpu.async_copy(w_v[b], acc_sp.at[dst_v[b]], ssem[b], add=True)

    def wait_scatter(k):
        b = k % 2
        pltpu.make_async_copy(w_v[b], acc_sp.at[dst_v[b]], ssem[b]).wait()

    start_idx(0)
    for k in range(NCHUNK):
        wait_idx(k)
        start_scatter(k)
        if k + 1 < NCHUNK:
            if k >= 1:
                wait_scatter(k - 1)
            start_idx(k + 1)
    wait_scatter(NCHUNK - 2)
    wait_scatter(NCHUNK - 1)

    plsc.subcore_barrier()

    @pl.when(s < 10)
    def _():
        pltpu.sync_copy(acc_sp.at[pl.ds(s * 1000, 1000)],
                        zbuf_v.at[pl.ds(0, 1000)])
        pltpu.sync_copy(zbuf_v.at[pl.ds(0, 1000)],
                        out_hbm.at[pl.ds(c * N + s * 1000, 1000)])


def _tc_prep_body(x_ref, w1_ref, degp_ref, h1_ref, p1_ref, dinv_ref):
    h = jnp.dot(x_ref[:], w1_ref[:], preferred_element_type=jnp.float32)
    deg = degp_ref[0, :] + degp_ref[1, :] + 1.0
    dinv = lax.rsqrt(deg)
    dinv2d = jnp.broadcast_to(dinv[:, None], (N, H))
    h1_ref[:] = h
    p1_ref[:] = dinv2d * h
    dinv_ref[:] = dinv2d


def _tc_mid_body(aggp_ref, h1_ref, dinv_ref, b1_ref, r_ref, p2_ref):
    dinv = dinv_ref[:]
    agg = aggp_ref[0] + aggp_ref[1]
    s1 = dinv * agg + dinv * dinv * h1_ref[:] + b1_ref[:]
    r = jnp.maximum(s1, 0.0)
    r_ref[:] = r
    p2_ref[:] = dinv * r


def _tc_out_body(aggp_ref, r_ref, dinv_ref, w2_ref, b2_ref, o_ref):
    dinv = dinv_ref[:]
    agg = aggp_ref[0] + aggp_ref[1]
    s2 = dinv * agg + dinv * dinv * r_ref[:]
    z = jnp.dot(s2, w2_ref[:], preferred_element_type=jnp.float32) + b2_ref[:]
    m = jnp.max(z, axis=1, keepdims=True)
    ez = jnp.exp(z - m)
    lse = jnp.log(jnp.sum(ez, axis=1, keepdims=True)) + m
    o_ref[:] = z - lse


_tc_prep = pl.pallas_call(
    _tc_prep_body,
    out_shape=(
        jax.ShapeDtypeStruct((N, H), jnp.float32),
        jax.ShapeDtypeStruct((N, H), jnp.float32),
        jax.ShapeDtypeStruct((N, H), jnp.float32),
    ),
)

_tc_mid = pl.pallas_call(
    _tc_mid_body,
    out_shape=(
        jax.ShapeDtypeStruct((N, H), jnp.float32),
        jax.ShapeDtypeStruct((N, H), jnp.float32),
    ),
)

_tc_out = pl.pallas_call(
    _tc_out_body,
    out_shape=jax.ShapeDtypeStruct((N, C), jnp.float32),
)


@jax.jit
def kernel(x, edge_index, edge_attr, W1, b1, W2, b2):
    src = edge_index[0]
    dst = edge_index[1]
    degp = _sc_degree(dst, edge_attr).reshape(NC, N)
    h1, p1, dinv = _tc_prep(x, W1, degp)
    agg1 = _sc_aggregate(p1, src, dst, edge_attr).reshape(NC, N, H)
    r, p2 = _tc_mid(agg1, h1, dinv, b1.reshape(1, H))
    agg2 = _sc_aggregate(p2, src, dst, edge_attr).reshape(NC, N, H)
    return _tc_out(agg2, r, dinv, W2, b2.reshape(1, C))
